# baseline (speedup 1.0000x reference)
import functools

import jax
import jax.numpy as jnp
from jax import lax
from jax.experimental import pallas as pl
from jax.experimental.pallas import tpu as pltpu
from jax.experimental.pallas import tpu_sc as plsc

B, N, S = 8, 4096, 1024
D1, D2 = 64, 128
C_IN = D1 + D2
M0, M1 = 128, 128
P = B * N

TN = 1024
TP = 4096
NB = N // TN
PB = P // TP
SA = B * NB

NW = 32
QW = P // NW
CQ = 128
NCH = QW // CQ

_BIG = 1e30


def _k1(xyz1_ref, xyz2t_ref, idx_ref, w_ref):
    a = xyz1_ref[0]
    bt = xyz2t_ref[0]
    dist = jnp.zeros((TN, S), jnp.float32)
    for d in range(3):
        diff = a[:, d:d + 1] - bt[d:d + 1, :]
        dist = dist + diff * diff

    iotac = lax.broadcasted_iota(jnp.int32, (S, 8), 0).astype(jnp.float32)
    base = pl.program_id(0) * S
    dcur = dist
    ws = []
    idxs = []
    wsum = None
    for k in range(3):
        mk = jnp.min(dcur, axis=1, keepdims=True)
        eq = dcur == mk
        wk = 1.0 / (mk + 1e-8)
        ikf = jnp.dot(eq.astype(jnp.float32), iotac,
                      preferred_element_type=jnp.float32,
                      precision=lax.Precision.HIGHEST)[:, :1]
        if k < 2:
            dcur = jnp.where(eq, _BIG, dcur)
        ws.append(wk)
        idxs.append(ikf.astype(jnp.int32) + base)
        wsum = wk if k == 0 else wsum + wk

    idx_ref[0] = jnp.concatenate(idxs, axis=1)
    w_ref[0] = jnp.concatenate([w / wsum for w in ws], axis=1)


def _sc_interp(idx_hbm, w_hbm, table_hbm, out_hbm,
               idx_v0, idx_v1, w_v0, w_v1, rows_v0, rows_v1, acc_v,
               sem0, sem1):
    wid = lax.axis_index("s") * 2 + lax.axis_index("c")
    wbase = wid * QW
    idx_vs = (idx_v0, idx_v1)
    w_vs = (w_v0, w_v1)
    rows_vs = (rows_v0, rows_v1)
    sems = (sem0, sem1)

    def start(ch, slot):
        cb = (wbase + ch * CQ) * 3
        pltpu.sync_copy(idx_hbm.at[pl.ds(cb, 3 * CQ)], idx_vs[slot])
        pltpu.sync_copy(w_hbm.at[pl.ds(cb, 3 * CQ)],
                        w_vs[slot].at[pl.ds(0, 3 * CQ)])
        pltpu.async_copy(table_hbm.at[idx_vs[slot]], rows_vs[slot], sems[slot])

    def finish(ch, slot):
        pltpu.make_async_copy(table_hbm.at[idx_vs[slot]], rows_vs[slot],
                              sems[slot]).wait()
        w_v = w_vs[slot]
        rows_v = rows_vs[slot]

        def q_body(q):
            r = q * 3
            wvec = w_v[pl.ds(r, 16)]
            for j in range(D2 // 16):
                acc = rows_v[r, pl.ds(16 * j, 16)] * wvec[0]
                acc = acc + rows_v[r + 1, pl.ds(16 * j, 16)] * wvec[1]
                acc = acc + rows_v[r + 2, pl.ds(16 * j, 16)] * wvec[2]
                acc_v[q, pl.ds(16 * j, 16)] = acc

        pl.loop(0, CQ)(q_body)
        pltpu.sync_copy(acc_v, out_hbm.at[pl.ds(wbase + ch * CQ, CQ)])

    start(0, 0)

    def chunk_pair(cp):
        start(cp + 1, 1)
        finish(cp, 0)
        start(jnp.minimum(cp + 2, NCH - 1), 0)
        finish(cp + 1, 1)

    pl.loop(0, NCH, step=2)(chunk_pair)
    pltpu.make_async_copy(table_hbm.at[idx_v0], rows_v0, sem0).wait()


def _bn(h, s, ss, g, be):
    m = s * (1.0 / P)
    v = ss * (1.0 / P) - m * m
    rstd = lax.rsqrt(v + 1e-5)
    return (h - m) * (rstd * g) + be


def _k3(p1_ref, interp_ref, w1at_ref, w1bt_ref, b1_ref,
        g1_ref, be1_ref, w2t_ref, b2_ref, g2_ref, be2_ref,
        out_ref, h1s, s1_ref, ss1_ref, s2_ref, ss2_ref):
    t = pl.program_id(0)

    @pl.when(t == 0)
    def _():
        s1_ref[...] = jnp.zeros_like(s1_ref)
        ss1_ref[...] = jnp.zeros_like(ss1_ref)
        s2_ref[...] = jnp.zeros_like(s2_ref)
        ss2_ref[...] = jnp.zeros_like(ss2_ref)

    @pl.when(t < PB)
    def _phase_a():
        h1 = (jnp.dot(p1_ref[...], w1at_ref[...],
                      preferred_element_type=jnp.float32)
              + jnp.dot(interp_ref[...], w1bt_ref[...],
                        preferred_element_type=jnp.float32)
              + b1_ref[...])
        h1s[pl.ds(pl.multiple_of(t * TP, TP), TP), :] = h1
        s1_ref[...] += jnp.sum(h1, axis=0, keepdims=True)
        ss1_ref[...] += jnp.sum(h1 * h1, axis=0, keepdims=True)

    @pl.when((t >= PB) & (t < 2 * PB))
    def _phase_b():
        off = pl.multiple_of((t - PB) * TP, TP)
        h1 = h1s[pl.ds(off, TP), :]
        a1 = jnp.maximum(_bn(h1, s1_ref[...], ss1_ref[...],
                             g1_ref[...], be1_ref[...]), 0.0)
        h2 = jnp.dot(a1, w2t_ref[...],
                     preferred_element_type=jnp.float32) + b2_ref[...]
        s2_ref[...] += jnp.sum(h2, axis=0, keepdims=True)
        ss2_ref[...] += jnp.sum(h2 * h2, axis=0, keepdims=True)

    @pl.when(t >= 2 * PB)
    def _phase_c():
        off = pl.multiple_of((t - 2 * PB) * TP, TP)
        h1 = h1s[pl.ds(off, TP), :]
        a1 = jnp.maximum(_bn(h1, s1_ref[...], ss1_ref[...],
                             g1_ref[...], be1_ref[...]), 0.0)
        h2 = jnp.dot(a1, w2t_ref[...],
                     preferred_element_type=jnp.float32) + b2_ref[...]
        out_ref[...] = jnp.maximum(_bn(h2, s2_ref[...], ss2_ref[...],
                                       g2_ref[...], be2_ref[...]), 0.0)


def kernel(xyz1, xyz2, points1, points2, W1, b1, g1, be1, W2, b2, g2, be2):
    xyz2t = jnp.transpose(xyz2, (0, 2, 1))
    w1at = jnp.transpose(W1[:, :D1])
    w1bt = jnp.transpose(W1[:, D1:])
    w2t = jnp.transpose(W2)
    row = lambda v: v.reshape(1, -1)

    idx3, w3 = pl.pallas_call(
        _k1,
        grid=(B, NB),
        in_specs=[
            pl.BlockSpec((1, TN, 3), lambda b, n: (b, n, 0)),
            pl.BlockSpec((1, 3, S), lambda b, n: (b, 0, 0)),
        ],
        out_specs=[
            pl.BlockSpec((1, TN, 3), lambda b, n: (b, n, 0)),
            pl.BlockSpec((1, TN, 3), lambda b, n: (b, n, 0)),
        ],
        out_shape=[
            jax.ShapeDtypeStruct((B, N, 3), jnp.int32),
            jax.ShapeDtypeStruct((B, N, 3), jnp.float32),
        ],
    )(xyz1, xyz2t)

    idx_flat = idx3.reshape(3 * P)
    w_flat = w3.reshape(3 * P)
    table = points2.reshape(B * S, D2)

    mesh = plsc.VectorSubcoreMesh(core_axis_name="c", subcore_axis_name="s")
    interp = functools.partial(
        pl.kernel,
        mesh=mesh,
        out_type=jax.ShapeDtypeStruct((P, D2), jnp.float32),
        scratch_types=[
            pltpu.VMEM((3 * CQ,), jnp.int32),
            pltpu.VMEM((3 * CQ,), jnp.int32),
            pltpu.VMEM((3 * CQ + 16,), jnp.float32),
            pltpu.VMEM((3 * CQ + 16,), jnp.float32),
            pltpu.VMEM((3 * CQ, D2), jnp.float32),
            pltpu.VMEM((3 * CQ, D2), jnp.float32),
            pltpu.VMEM((CQ, D2), jnp.float32),
            pltpu.SemaphoreType.DMA,
            pltpu.SemaphoreType.DMA,
        ],
    )(_sc_interp)(idx_flat, w_flat, table)

    p1f = points1.reshape(P, D1)

    out = pl.pallas_call(
        _k3,
        grid=(3 * PB,),
        in_specs=[
            pl.BlockSpec((TP, D1), lambda t: (jnp.minimum(t, PB - 1), 0)),
            pl.BlockSpec((TP, D2), lambda t: (jnp.minimum(t, PB - 1), 0)),
            pl.BlockSpec((D1, M0), lambda t: (0, 0)),
            pl.BlockSpec((D2, M0), lambda t: (0, 0)),
            pl.BlockSpec((1, M0), lambda t: (0, 0)),
            pl.BlockSpec((1, M0), lambda t: (0, 0)),
            pl.BlockSpec((1, M0), lambda t: (0, 0)),
            pl.BlockSpec((M0, M1), lambda t: (0, 0)),
            pl.BlockSpec((1, M1), lambda t: (0, 0)),
            pl.BlockSpec((1, M1), lambda t: (0, 0)),
            pl.BlockSpec((1, M1), lambda t: (0, 0)),
        ],
        out_specs=pl.BlockSpec(
            (TP, M1), lambda t: (jnp.maximum(t - 2 * PB, 0), 0)),
        out_shape=jax.ShapeDtypeStruct((P, M1), jnp.float32),
        scratch_shapes=[
            pltpu.VMEM((P, M0), jnp.float32),
            pltpu.VMEM((1, M0), jnp.float32),
            pltpu.VMEM((1, M0), jnp.float32),
            pltpu.VMEM((1, M1), jnp.float32),
            pltpu.VMEM((1, M1), jnp.float32),
        ],
    )(p1f, interp, w1at, w1bt, row(b1), row(g1), row(be1),
      w2t, row(b2), row(g2), row(be2))

    return out.reshape(B, N, M1)


# speedup vs baseline: 1.8773x; 1.8773x over previous
import functools

import jax
import jax.numpy as jnp
from jax import lax
from jax.experimental import pallas as pl
from jax.experimental.pallas import tpu as pltpu
from jax.experimental.pallas import tpu_sc as plsc

B, N, S = 8, 4096, 1024
D1, D2 = 64, 128
C_IN = D1 + D2
M0, M1 = 128, 128
P = B * N

TN = 1024
TP = 4096
NB = N // TN
PB = P // TP
SA = B * NB

NW = 32
QW = P // NW
CQ = 128
NCH = QW // CQ

_BIG = 1e30


def _k1(xyz1_ref, xyz2t_ref, idx_ref, w_ref):
    a = xyz1_ref[0]
    bt = xyz2t_ref[0]
    dist = jnp.zeros((TN, S), jnp.float32)
    for d in range(3):
        diff = a[:, d:d + 1] - bt[d:d + 1, :]
        dist = dist + diff * diff

    iota = lax.broadcasted_iota(jnp.int32, (TN, S), 1)
    base = pl.program_id(0) * S
    dcur = dist
    ws = []
    idxs = []
    wsum = None
    for k in range(3):
        mk = jnp.min(dcur, axis=1, keepdims=True)
        eq = dcur == mk
        ik = jnp.min(jnp.where(eq, iota, S), axis=1, keepdims=True)
        wk = 1.0 / (mk + 1e-8)
        if k < 2:
            dcur = jnp.where(iota == ik, _BIG, dcur)
        ws.append(wk)
        idxs.append(ik + base)
        wsum = wk if k == 0 else wsum + wk

    idx_ref[0] = jnp.concatenate(idxs, axis=1)
    w_ref[0] = jnp.concatenate([w / wsum for w in ws], axis=1)


def _sc_interp(idx_hbm, w_hbm, table_hbm, out_hbm,
               idx_v0, idx_v1, w_v0, w_v1, rows_v0, rows_v1, acc_v,
               sem0, sem1):
    wid = lax.axis_index("s") * 2 + lax.axis_index("c")
    wbase = wid * QW
    idx_vs = (idx_v0, idx_v1)
    w_vs = (w_v0, w_v1)
    rows_vs = (rows_v0, rows_v1)
    sems = (sem0, sem1)

    def start(ch, slot):
        cb = (wbase + ch * CQ) * 3
        pltpu.sync_copy(idx_hbm.at[pl.ds(cb, 3 * CQ)], idx_vs[slot])
        pltpu.sync_copy(w_hbm.at[pl.ds(cb, 3 * CQ)],
                        w_vs[slot].at[pl.ds(0, 3 * CQ)])
        pltpu.async_copy(table_hbm.at[idx_vs[slot]], rows_vs[slot], sems[slot])

    def finish(ch, slot):
        pltpu.make_async_copy(table_hbm.at[idx_vs[slot]], rows_vs[slot],
                              sems[slot]).wait()
        w_v = w_vs[slot]
        rows_v = rows_vs[slot]

        def q_body(q):
            r = q * 3
            wvec = w_v[pl.ds(r, 16)]
            for j in range(D2 // 16):
                acc = rows_v[r, pl.ds(16 * j, 16)] * wvec[0]
                acc = acc + rows_v[r + 1, pl.ds(16 * j, 16)] * wvec[1]
                acc = acc + rows_v[r + 2, pl.ds(16 * j, 16)] * wvec[2]
                acc_v[q, pl.ds(16 * j, 16)] = acc

        pl.loop(0, CQ)(q_body)
        pltpu.sync_copy(acc_v, out_hbm.at[pl.ds(wbase + ch * CQ, CQ)])

    start(0, 0)

    def chunk_pair(cp):
        start(cp + 1, 1)
        finish(cp, 0)
        start(jnp.minimum(cp + 2, NCH - 1), 0)
        finish(cp + 1, 1)

    pl.loop(0, NCH, step=2)(chunk_pair)
    pltpu.make_async_copy(table_hbm.at[idx_v0], rows_v0, sem0).wait()


def _bn(h, s, ss, g, be):
    m = s * (1.0 / P)
    v = ss * (1.0 / P) - m * m
    rstd = lax.rsqrt(v + 1e-5)
    return (h - m) * (rstd * g) + be


def _k3(p1_ref, interp_ref, w1at_ref, w1bt_ref, b1_ref,
        g1_ref, be1_ref, w2t_ref, b2_ref, g2_ref, be2_ref,
        out_ref, h1s, s1_ref, ss1_ref, s2_ref, ss2_ref):
    t = pl.program_id(0)

    @pl.when(t == 0)
    def _():
        s1_ref[...] = jnp.zeros_like(s1_ref)
        ss1_ref[...] = jnp.zeros_like(ss1_ref)
        s2_ref[...] = jnp.zeros_like(s2_ref)
        ss2_ref[...] = jnp.zeros_like(ss2_ref)

    @pl.when(t < PB)
    def _phase_a():
        h1 = (jnp.dot(p1_ref[...], w1at_ref[...],
                      preferred_element_type=jnp.float32)
              + jnp.dot(interp_ref[...], w1bt_ref[...],
                        preferred_element_type=jnp.float32)
              + b1_ref[...])
        h1s[pl.ds(pl.multiple_of(t * TP, TP), TP), :] = h1
        s1_ref[...] += jnp.sum(h1, axis=0, keepdims=True)
        ss1_ref[...] += jnp.sum(h1 * h1, axis=0, keepdims=True)

    @pl.when((t >= PB) & (t < 2 * PB))
    def _phase_b():
        off = pl.multiple_of((t - PB) * TP, TP)
        h1 = h1s[pl.ds(off, TP), :]
        a1 = jnp.maximum(_bn(h1, s1_ref[...], ss1_ref[...],
                             g1_ref[...], be1_ref[...]), 0.0)
        h2 = jnp.dot(a1, w2t_ref[...],
                     preferred_element_type=jnp.float32) + b2_ref[...]
        s2_ref[...] += jnp.sum(h2, axis=0, keepdims=True)
        ss2_ref[...] += jnp.sum(h2 * h2, axis=0, keepdims=True)

    @pl.when(t >= 2 * PB)
    def _phase_c():
        off = pl.multiple_of((t - 2 * PB) * TP, TP)
        h1 = h1s[pl.ds(off, TP), :]
        a1 = jnp.maximum(_bn(h1, s1_ref[...], ss1_ref[...],
                             g1_ref[...], be1_ref[...]), 0.0)
        h2 = jnp.dot(a1, w2t_ref[...],
                     preferred_element_type=jnp.float32) + b2_ref[...]
        out_ref[...] = jnp.maximum(_bn(h2, s2_ref[...], ss2_ref[...],
                                       g2_ref[...], be2_ref[...]), 0.0)


def kernel(xyz1, xyz2, points1, points2, W1, b1, g1, be1, W2, b2, g2, be2):
    xyz2t = jnp.transpose(xyz2, (0, 2, 1))
    w1at = jnp.transpose(W1[:, :D1])
    w1bt = jnp.transpose(W1[:, D1:])
    w2t = jnp.transpose(W2)
    row = lambda v: v.reshape(1, -1)

    idx3, w3 = pl.pallas_call(
        _k1,
        grid=(B, NB),
        in_specs=[
            pl.BlockSpec((1, TN, 3), lambda b, n: (b, n, 0)),
            pl.BlockSpec((1, 3, S), lambda b, n: (b, 0, 0)),
        ],
        out_specs=[
            pl.BlockSpec((1, TN, 3), lambda b, n: (b, n, 0)),
            pl.BlockSpec((1, TN, 3), lambda b, n: (b, n, 0)),
        ],
        out_shape=[
            jax.ShapeDtypeStruct((B, N, 3), jnp.int32),
            jax.ShapeDtypeStruct((B, N, 3), jnp.float32),
        ],
    )(xyz1, xyz2t)

    idx_flat = idx3.reshape(3 * P)
    w_flat = w3.reshape(3 * P)
    table = points2.reshape(B * S, D2)

    mesh = plsc.VectorSubcoreMesh(core_axis_name="c", subcore_axis_name="s")
    interp = functools.partial(
        pl.kernel,
        mesh=mesh,
        out_type=jax.ShapeDtypeStruct((P, D2), jnp.float32),
        scratch_types=[
            pltpu.VMEM((3 * CQ,), jnp.int32),
            pltpu.VMEM((3 * CQ,), jnp.int32),
            pltpu.VMEM((3 * CQ + 16,), jnp.float32),
            pltpu.VMEM((3 * CQ + 16,), jnp.float32),
            pltpu.VMEM((3 * CQ, D2), jnp.float32),
            pltpu.VMEM((3 * CQ, D2), jnp.float32),
            pltpu.VMEM((CQ, D2), jnp.float32),
            pltpu.SemaphoreType.DMA,
            pltpu.SemaphoreType.DMA,
        ],
    )(_sc_interp)(idx_flat, w_flat, table)

    p1f = points1.reshape(P, D1)

    out = pl.pallas_call(
        _k3,
        grid=(3 * PB,),
        in_specs=[
            pl.BlockSpec((TP, D1), lambda t: (jnp.minimum(t, PB - 1), 0)),
            pl.BlockSpec((TP, D2), lambda t: (jnp.minimum(t, PB - 1), 0)),
            pl.BlockSpec((D1, M0), lambda t: (0, 0)),
            pl.BlockSpec((D2, M0), lambda t: (0, 0)),
            pl.BlockSpec((1, M0), lambda t: (0, 0)),
            pl.BlockSpec((1, M0), lambda t: (0, 0)),
            pl.BlockSpec((1, M0), lambda t: (0, 0)),
            pl.BlockSpec((M0, M1), lambda t: (0, 0)),
            pl.BlockSpec((1, M1), lambda t: (0, 0)),
            pl.BlockSpec((1, M1), lambda t: (0, 0)),
            pl.BlockSpec((1, M1), lambda t: (0, 0)),
        ],
        out_specs=pl.BlockSpec(
            (TP, M1), lambda t: (jnp.maximum(t - 2 * PB, 0), 0)),
        out_shape=jax.ShapeDtypeStruct((P, M1), jnp.float32),
        scratch_shapes=[
            pltpu.VMEM((P, M0), jnp.float32),
            pltpu.VMEM((1, M0), jnp.float32),
            pltpu.VMEM((1, M0), jnp.float32),
            pltpu.VMEM((1, M1), jnp.float32),
            pltpu.VMEM((1, M1), jnp.float32),
        ],
    )(p1f, interp, w1at, w1bt, row(b1), row(g1), row(be1),
      w2t, row(b2), row(g2), row(be2))

    return out.reshape(B, N, M1)
